# Initial kernel scaffold; baseline (speedup 1.0000x reference)
import jax
import jax.numpy as jnp
from jax.experimental import pallas as pl


def kernel(x, u, batch, W1, b1, W2, b2, W3, b3, ln1_w, ln1_b, W4, b4, W5, b5, W6, b6, ln2_w, ln2_b):
    raise NotImplementedError("write your pallas kernel here")


# speedup vs baseline: 10.5210x; 10.5210x over previous
import jax
import jax.numpy as jnp
from jax.experimental import pallas as pl
from jax.experimental.pallas import tpu as pltpu

N = 100000
B = 64
D = 128
BLK = 4000
NB = N // BLK


def _ln(h, w, b):
    m = jnp.mean(h, axis=-1, keepdims=True)
    v = jnp.mean((h - m) ** 2, axis=-1, keepdims=True)
    return (h - m) * jax.lax.rsqrt(v + 1e-5) * w + b


def _dot(a, b):
    return jnp.dot(a, b, preferred_element_type=jnp.float32)


def _fused(x_ref, batch_ref, u_ref, W1_ref, b1_ref, W2_ref, b2_ref,
           W3_ref, b3_ref, ln1w_ref, ln1b_ref, W4_ref, b4_ref, W5_ref,
           b5_ref, W6_ref, b6_ref, ln2w_ref, ln2b_ref, out_ref,
           acc_ref, uproj_ref):
    i = pl.program_id(0)

    @pl.when(i == 0)
    def _init():
        uproj_ref[...] = _dot(u_ref[...], W1_ref[D:, :])
        acc_ref[...] = jnp.zeros_like(acc_ref)

    ids = batch_ref[0, 0, :]
    onehot = (ids[:, None] ==
              jax.lax.broadcasted_iota(jnp.int32, (BLK, B), 1)
              ).astype(jnp.float32)
    h = _dot(x_ref[...], W1_ref[:D, :]) + _dot(onehot, uproj_ref[...])
    h = jnp.maximum(h + b1_ref[...], 0.0)
    h = jnp.maximum(_dot(h, W2_ref[...]) + b2_ref[...], 0.0)
    h = _dot(h, W3_ref[...]) + b3_ref[...]
    h = _ln(h, ln1w_ref[...], ln1b_ref[...])
    acc_ref[...] += jax.lax.dot_general(
        onehot, h, (((0,), (0,)), ((), ())),
        preferred_element_type=jnp.float32)

    @pl.when(i == NB - 1)
    def _finish():
        agg = acc_ref[...]
        uu = u_ref[...]
        h2 = _dot(agg, W4_ref[:D, :]) + _dot(uu, W4_ref[D:, :])
        h2 = jnp.maximum(h2 + b4_ref[...], 0.0)
        h2 = jnp.maximum(_dot(h2, W5_ref[...]) + b5_ref[...], 0.0)
        h2 = _dot(h2, W6_ref[...]) + b6_ref[...]
        h2 = _ln(h2, ln2w_ref[...], ln2b_ref[...])
        out_ref[...] = h2 + uu


def kernel(x, u, batch, W1, b1, W2, b2, W3, b3, ln1_w, ln1_b,
           W4, b4, W5, b5, W6, b6, ln2_w, ln2_b):
    batch3 = batch.reshape(NB, 1, BLK)
    row = lambda v: v.reshape(1, D)

    def fixed(shape):
        return pl.BlockSpec(shape, lambda i: (0,) * len(shape))

    in_specs = [
            pl.BlockSpec((BLK, D), lambda i: (i, 0)),
            pl.BlockSpec((1, 1, BLK), lambda i: (i, 0, 0)),
            fixed((B, D)),
            fixed((2 * D, D)),
            fixed((1, D)),
            fixed((D, D)), fixed((1, D)),
            fixed((D, D)), fixed((1, D)),
            fixed((1, D)), fixed((1, D)),
            fixed((2 * D, D)), fixed((1, D)),
            fixed((D, D)), fixed((1, D)),
            fixed((D, D)), fixed((1, D)),
            fixed((1, D)), fixed((1, D)),
        ]
    return pl.pallas_call(
        _fused,
        grid=(NB,),
        in_specs=in_specs,
        out_specs=fixed((B, D)),
        out_shape=jax.ShapeDtypeStruct((B, D), jnp.float32),
        scratch_shapes=[pltpu.VMEM((B, D), jnp.float32),
                        pltpu.VMEM((B, D), jnp.float32)],
    )(x, batch3, u, W1, row(b1), W2, row(b2), W3, row(b3),
      row(ln1_w), row(ln1_b), W4, row(b4), W5, row(b5), W6, row(b6),
      row(ln2_w), row(ln2_b))
